# baseline (speedup 1.0000x reference)
import functools
import math

import jax
import jax.numpy as jnp
from jax import lax
from jax.experimental import pallas as pl
from jax.experimental.pallas import tpu as pltpu
from jax.experimental.pallas import tpu_sc as plsc

VOCAB = 1000000
EMBED = 64
WINDOW = 200
BATCH = 4096

NUM_CORES = 2
NUM_SUBCORES = 16
NUM_WORKERS = NUM_CORES * NUM_SUBCORES

UNIT = 100
TOTAL = BATCH * WINDOW
UNITS_PER_WORKER = TOTAL // (NUM_WORKERS * UNIT)
IDX_BLOCK = 16
SCALE = math.sqrt(EMBED)


def _sc_embed(x2, table, pos_enc):
    mesh = plsc.VectorSubcoreMesh(core_axis_name="c", subcore_axis_name="s")

    @functools.partial(
        pl.kernel,
        mesh=mesh,
        compiler_params=pltpu.CompilerParams(use_tc_tiling_on_sc=False),
        out_type=jax.ShapeDtypeStruct((TOTAL // UNIT, UNIT, EMBED),
                                      jnp.float32),
        scratch_types=[
            pltpu.VMEM((IDX_BLOCK, UNIT), jnp.int32),
            pltpu.VMEM((WINDOW, EMBED), jnp.float32),
            pltpu.VMEM((2, UNIT, EMBED), jnp.float32),
            pltpu.SemaphoreType.DMA,
        ],
    )
    def k(x_hbm, table_hbm, pos_hbm, out_hbm, idx_v, pos_v, buf, sem):
        wid = lax.axis_index("s") * NUM_CORES + lax.axis_index("c")
        base_u = wid * UNITS_PER_WORKER

        pltpu.sync_copy(pos_hbm, pos_v)

        def outer(g, _):
            pltpu.sync_copy(x_hbm.at[pl.ds(base_u + g * IDX_BLOCK, IDX_BLOCK)],
                            idx_v)

            def pair(h, _):
                for p in range(2):
                    j = h * 2 + p
                    u = base_u + g * IDX_BLOCK + j
                    pltpu.async_copy(table_hbm.at[idx_v.at[j]], buf.at[p],
                                     sem).wait()

                    def rows(r, _):
                        for q in range(EMBED // 16):
                            e = buf[p, r, pl.ds(q * 16, 16)]
                            po = pos_v[p * UNIT + r, pl.ds(q * 16, 16)]
                            buf[p, r, pl.ds(q * 16, 16)] = e * SCALE + po
                        return 0

                    lax.fori_loop(0, UNIT, rows, 0)
                    pltpu.sync_copy(buf.at[p], out_hbm.at[u])
                return 0

            lax.fori_loop(0, IDX_BLOCK // 2, pair, 0)
            return 0

        lax.fori_loop(0, UNITS_PER_WORKER // IDX_BLOCK, outer, 0)

    return k(x2, table, pos_enc)


def kernel(x, table, pos_enc):
    x2 = x.reshape(TOTAL // UNIT, UNIT).astype(jnp.int32)
    out = _sc_embed(x2, table, pos_enc)
    return out.reshape(BATCH, WINDOW, EMBED)


# speedup vs baseline: 1.2063x; 1.2063x over previous
import functools
import math

import jax
import jax.numpy as jnp
from jax import lax
from jax.experimental import pallas as pl
from jax.experimental.pallas import tpu as pltpu
from jax.experimental.pallas import tpu_sc as plsc

VOCAB = 1000000
EMBED = 64
WINDOW = 200
BATCH = 4096

NUM_CORES = 2
NUM_SUBCORES = 16
NUM_WORKERS = NUM_CORES * NUM_SUBCORES

UNIT = 100
TOTAL = BATCH * WINDOW
UNITS_PER_WORKER = TOTAL // (NUM_WORKERS * UNIT)
NBUF = 4
SCALE = math.sqrt(EMBED)


def _sc_embed(x2, table, pos_enc):
    mesh = plsc.VectorSubcoreMesh(core_axis_name="c", subcore_axis_name="s")

    @functools.partial(
        pl.kernel,
        mesh=mesh,
        compiler_params=pltpu.CompilerParams(use_tc_tiling_on_sc=False),
        out_type=jax.ShapeDtypeStruct((TOTAL // UNIT, UNIT, EMBED),
                                      jnp.float32),
        scratch_types=[
            pltpu.VMEM((UNITS_PER_WORKER, UNIT), jnp.int32),
            pltpu.VMEM((WINDOW, EMBED), jnp.float32),
            pltpu.VMEM((NBUF, UNIT, EMBED), jnp.float32),
            pltpu.SemaphoreType.DMA,
            pltpu.SemaphoreType.DMA,
        ],
    )
    def k(x_hbm, table_hbm, pos_hbm, out_hbm, idx_all, pos_v, buf,
          sem_g, sem_s):
        wid = lax.axis_index("s") * NUM_CORES + lax.axis_index("c")
        base_u = wid * UNITS_PER_WORKER

        pltpu.sync_copy(pos_hbm, pos_v)
        pltpu.sync_copy(x_hbm.at[pl.ds(base_u, UNITS_PER_WORKER)], idx_all)

        def start_gather(u, b):
            pltpu.async_copy(table_hbm.at[idx_all.at[u]], buf.at[b], sem_g)

        def wait_gather(b):
            pltpu.make_async_copy(table_hbm.at[pl.ds(0, UNIT)], buf.at[b],
                                  sem_g).wait()

        def start_scatter(u, b):
            pltpu.async_copy(buf.at[b], out_hbm.at[base_u + u], sem_s)

        def wait_scatter(b):
            pltpu.make_async_copy(buf.at[b], out_hbm.at[0], sem_s).wait()

        def compute(b, p):
            def rows(rr, _):
                for dr in range(2):
                    r = rr * 2 + dr
                    for q in range(EMBED // 16):
                        e = buf[b, r, pl.ds(q * 16, 16)]
                        po = pos_v[p * UNIT + r, pl.ds(q * 16, 16)]
                        buf[b, r, pl.ds(q * 16, 16)] = e * SCALE + po
                return 0

            lax.fori_loop(0, UNIT // 2, rows, 0)

        start_gather(0, 0)
        start_gather(1, 1)

        for b in range(NBUF):
            wait_gather(b)
            compute(b, b % 2)
            start_scatter(b, b)
            if b < 2:
                start_gather(b + 2, b + 2)
            else:
                wait_scatter(b - 2)
                start_gather(b + 2, b - 2)

        def body(i, _):
            u0 = i * NBUF
            for b in range(NBUF):
                u = u0 + b
                wait_gather(b)
                compute(b, b % 2)
                start_scatter(u, b)
                wait_scatter(b)
                start_gather(u + 2, (b + 2) % NBUF)
            return 0

        lax.fori_loop(1, UNITS_PER_WORKER // NBUF - 1, body, 0)

        u0 = UNITS_PER_WORKER - NBUF
        for b in range(NBUF):
            u = u0 + b
            wait_gather(b)
            compute(b, b % 2)
            start_scatter(u, b)
            if b < 2:
                wait_scatter(b)
                start_gather(u + 2, (b + 2) % NBUF)

        for b in range(NBUF):
            wait_scatter(b)

    return k(x2, table, pos_enc)


def kernel(x, table, pos_enc):
    x2 = x.reshape(TOTAL // UNIT, UNIT).astype(jnp.int32)
    out = _sc_embed(x2, table, pos_enc)
    return out.reshape(BATCH, WINDOW, EMBED)
